# baseline (speedup 1.0000x reference)
import functools

import jax
import jax.numpy as jnp
from jax.experimental import pallas as pl
from jax.experimental.pallas import tpu as pltpu


def _body(x_ref, b_ref, W1_ref, b1_ref, gW2_ref, gb2_ref, fW2_ref, fb2_ref,
          out_ref, m_ref, d_ref, S_ref, *, R, P, Bn, DH, DE):
    i = pl.program_id(0)
    L = P * Bn

    @pl.when(i == 0)
    def _init():
        m_ref[...] = jnp.full((1, L), -1e30, jnp.float32)
        d_ref[...] = jnp.zeros((1, L), jnp.float32)
        S_ref[...] = jnp.zeros((L, P * DE), jnp.float32)

    xb = x_ref[...].astype(jnp.bfloat16)
    h = jax.lax.dot_general(xb, W1_ref[...], (((1,), (0,)), ((), ())),
                            preferred_element_type=jnp.float32)
    h = jnp.maximum(h.astype(jnp.bfloat16) + b1_ref[...],
                    jnp.bfloat16(0))

    gboth = jax.lax.dot_general(h[:, :P * DH], gW2_ref[...],
                                (((1,), (0,)), ((), ())),
                                preferred_element_type=jnp.float32)
    gboth = gboth + gb2_ref[...]

    bb = b_ref[...]
    seg_ids = jax.lax.broadcasted_iota(jnp.int32, (R, L), 1) & (Bn - 1)
    O = bb == seg_ids

    masked = jnp.where(O, gboth, -1e30)
    bmax = jnp.max(masked, axis=0, keepdims=True)
    m_old = m_ref[...]
    m_new = jnp.maximum(m_old, bmax)
    scale = jnp.exp(m_old - m_new)
    E = jnp.where(O, jnp.exp(gboth - m_new), 0.0)
    d_ref[...] = d_ref[...] * scale + jnp.sum(E, axis=0, keepdims=True)
    m_ref[...] = m_new

    f0 = jax.lax.dot_general(h[:, P * DH:(P + 1) * DH], fW2_ref[0],
                             (((1,), (0,)), ((), ())),
                             preferred_element_type=jnp.float32)
    f1 = jax.lax.dot_general(h[:, (P + 1) * DH:], fW2_ref[1],
                             (((1,), (0,)), ((), ())),
                             preferred_element_type=jnp.float32)
    Fcat = jnp.concatenate([f0, f1], axis=1).astype(jnp.bfloat16)
    Fcat = Fcat + fb2_ref[...]
    S_ref[...] = (S_ref[...] * jnp.transpose(scale)
                  + jax.lax.dot_general(E.astype(jnp.bfloat16), Fcat,
                                        (((0,), (0,)), ((), ())),
                                        preferred_element_type=jnp.float32))

    @pl.when(i == pl.num_programs(0) - 1)
    def _finish():
        for k in range(P):
            dT = jnp.transpose(d_ref[:, k * Bn:(k + 1) * Bn])
            Sk = S_ref[k * Bn:(k + 1) * Bn, k * DE:(k + 1) * DE]
            out_ref[k] = jnp.where(dT > 0.0, Sk / dT, 0.0)


def kernel(x, batch, n_nodes, Omegas, Phis, Lambdas, Omegas_norm, Phis_norm,
           Lambdas_norm, gate_W1, gate_b1, gate_W2, gate_b2, feat_W1, feat_b1,
           feat_W2, feat_b2):
    N, FD = x.shape
    Bn = n_nodes.shape[0]
    P, _, DH = gate_W1.shape
    DE = feat_W2.shape[2]
    R = 5000
    assert N % R == 0

    W1all = jnp.concatenate(
        [gate_W1[k] for k in range(P)] + [feat_W1[k] for k in range(P)],
        axis=1).astype(jnp.bfloat16)
    b1all = jnp.concatenate(
        [gate_b1[k] for k in range(P)]
        + [feat_b1[k] for k in range(P)])[None, :].astype(jnp.bfloat16)
    gW2w = jnp.zeros((P * DH, P * Bn), jnp.float32)
    gb2w = jnp.zeros((1, P * Bn), jnp.float32)
    for k in range(P):
        gW2w = gW2w.at[k * DH:(k + 1) * DH, k * Bn:(k + 1) * Bn].set(
            jnp.tile(gate_W2[k, :, 0:1], (1, Bn)))
        gb2w = gb2w.at[0, k * Bn:(k + 1) * Bn].set(gate_b2[k, 0])
    gW2w = gW2w.astype(jnp.bfloat16)
    fW2b = feat_W2.astype(jnp.bfloat16)
    fb2c = jnp.concatenate([feat_b2[k] for k in range(P)])[None, :].astype(
        jnp.bfloat16)
    batch2 = batch.astype(jnp.int32).reshape(N, 1)

    body = functools.partial(_body, R=R, P=P, Bn=Bn, DH=DH, DE=DE)
    pools = pl.pallas_call(
        body,
        grid=(N // R,),
        in_specs=[
            pl.BlockSpec((R, FD), lambda i: (i, 0)),
            pl.BlockSpec((R, 1), lambda i: (i, 0)),
            pl.BlockSpec((FD, 2 * P * DH), lambda i: (0, 0)),
            pl.BlockSpec((1, 2 * P * DH), lambda i: (0, 0)),
            pl.BlockSpec((P * DH, P * Bn), lambda i: (0, 0)),
            pl.BlockSpec((1, P * Bn), lambda i: (0, 0)),
            pl.BlockSpec((P, DH, DE), lambda i: (0, 0, 0)),
            pl.BlockSpec((1, P * DE), lambda i: (0, 0)),
        ],
        out_specs=pl.BlockSpec((P, Bn, DE), lambda i: (0, 0, 0)),
        out_shape=jax.ShapeDtypeStruct((P, Bn, DE), jnp.float32),
        scratch_shapes=[
            pltpu.VMEM((1, P * Bn), jnp.float32),
            pltpu.VMEM((1, P * Bn), jnp.float32),
            pltpu.VMEM((P * Bn, P * DE), jnp.float32),
        ],
    )(x, batch2, W1all, b1all, gW2w, gb2w, fW2b, fb2c)

    return jnp.concatenate(
        [pools[k] for k in range(P)]
        + [n_nodes, Omegas, Phis, Lambdas, Omegas_norm, Phis_norm,
           Lambdas_norm], axis=1)
